# baseline (speedup 1.0000x reference)
import jax
import jax.numpy as jnp
from jax.experimental import pallas as pl
from jax.experimental.pallas import tpu as pltpu

B, N, C = 2, 2048, 768
E, H, D, K = 8, 384, 4, 2

_NEG_INF = float("-inf")
_CH = 256
_NCH = N // _CH


def _gelu(v):
    return v * 0.5 * (1.0 + jax.lax.erf(v * 0.7071067811865476))


def _moe_kernel(tids_ref, x_hbm, gw_ref, epsT_ref, w1_hbm, b1_ref, w2_hbm,
                b2_ref, out_hbm, xv, outv, w1s, w2s, sx, sw, so):
    cps_x = []
    for b in range(B):
        for i in range(_NCH):
            cp = pltpu.make_async_copy(
                x_hbm.at[b, pl.ds(i * _CH, _CH), :],
                xv.at[b, pl.ds(i * _CH, _CH), :],
                sx.at[b * _NCH + i])
            cp.start()
            cps_x.append(cp)

    gate_info = []
    for b in range(B):
        tid = tids_ref[b]
        gwb = gw_ref[pl.ds(tid, 1), :, :][0]
        s = jnp.zeros((1, E), jnp.float32)
        for i in range(_NCH):
            cps_x[b * _NCH + i].wait()
            xc = xv[b, pl.ds(i * _CH, _CH), :]
            twc = jnp.dot(xc, gwb, preferred_element_type=jnp.float32)
            clean = twc[:, :E]
            stdc = jax.nn.softplus(twc[:, E:]) + 0.01
            epc = epsT_ref[b, :, pl.ds(i * _CH, _CH)]
            m = jnp.dot(epc, stdc, preferred_element_type=jnp.float32)
            ir = jax.lax.broadcasted_iota(jnp.int32, (E, E), 0)
            ic = jax.lax.broadcasted_iota(jnp.int32, (E, E), 1)
            diag = jnp.sum(jnp.where(ir == ic, m, 0.0), axis=0, keepdims=True)
            s = s + jnp.sum(clean, axis=0, keepdims=True) + diag
        iota = jax.lax.broadcasted_iota(jnp.int32, (1, E), 1)
        m2 = jnp.max(s)
        e0 = jnp.min(jnp.where(s == m2, iota, E))
        masked = jnp.where(iota == e0, _NEG_INF, s)
        m1 = jnp.max(masked)
        e1 = jnp.min(jnp.where(masked == m1, iota, E))
        d = m2 - m1
        a = d / (d + 1e-6)
        ena = jnp.exp(-a)
        g0 = 1.0 / (1.0 + ena)
        g1 = ena / (1.0 + ena)
        cws = []
        for j, e in enumerate((e0, e1)):
            cp1 = pltpu.make_async_copy(w1_hbm.at[e], w1s.at[b * K + j],
                                        sw.at[b * K + j])
            cp2 = pltpu.make_async_copy(w2_hbm.at[e], w2s.at[b * K + j],
                                        sw.at[B * K + b * K + j])
            cp1.start()
            cp2.start()
            cws.append((cp1, cp2))
        gate_info.append((e0, e1, g0, g1, cws))

    cps_o = []
    for b in range(B):
        e0, e1, g0, g1, cws = gate_info[b]
        cws[0][0].wait()
        cws[0][1].wait()
        cws[1][0].wait()
        cws[1][1].wait()
        w1cat = jnp.concatenate(
            [w1s[b * K + 0], w1s[b * K + 1]], axis=1).astype(jnp.bfloat16)
        w2cat = jnp.concatenate(
            [w2s[b * K + 0], w2s[b * K + 1]], axis=0).astype(jnp.bfloat16)
        b1cat = jnp.concatenate(
            [b1_ref[pl.ds(e0, 1), :], b1_ref[pl.ds(e1, 1), :]], axis=1)
        gsc = jnp.concatenate(
            [jnp.full((1, H), 1.0, jnp.float32) * g0,
             jnp.full((1, H), 1.0, jnp.float32) * g1], axis=1)
        b2mix = g0 * b2_ref[pl.ds(e0, 1), :] + g1 * b2_ref[pl.ds(e1, 1), :]
        for i in range(_NCH):
            xc = xv[b, pl.ds(i * _CH, _CH), :]
            xbf = xc.astype(jnp.bfloat16)
            h = jnp.dot(xbf, w1cat, preferred_element_type=jnp.float32) + b1cat
            hg = (_gelu(h) * gsc).astype(jnp.bfloat16)
            y = jnp.dot(hg, w2cat, preferred_element_type=jnp.float32) + b2mix
            outv[b, pl.ds(i * _CH, _CH), :] = xc + y
            cp = pltpu.make_async_copy(
                outv.at[b, pl.ds(i * _CH, _CH), :],
                out_hbm.at[b, pl.ds(i * _CH, _CH), :],
                so.at[b * _NCH + i])
            cp.start()
            cps_o.append(cp)

    for cp in cps_o:
        cp.wait()


@jax.jit
def kernel(x, gate_w, w1, b1, w2, b2, eps, task_ids):
    task_ids = task_ids.astype(jnp.int32)
    epsT = jnp.transpose(eps, (0, 2, 1))

    out = pl.pallas_call(
        _moe_kernel,
        grid_spec=pltpu.PrefetchScalarGridSpec(
            num_scalar_prefetch=1,
            grid=(1,),
            in_specs=[
                pl.BlockSpec(memory_space=pltpu.MemorySpace.HBM),
                pl.BlockSpec((D, C, 2 * E), lambda i, tids: (0, 0, 0)),
                pl.BlockSpec((B, E, N), lambda i, tids: (0, 0, 0)),
                pl.BlockSpec(memory_space=pltpu.MemorySpace.HBM),
                pl.BlockSpec((E, H), lambda i, tids: (0, 0)),
                pl.BlockSpec(memory_space=pltpu.MemorySpace.HBM),
                pl.BlockSpec((E, C), lambda i, tids: (0, 0)),
            ],
            out_specs=pl.BlockSpec(memory_space=pltpu.MemorySpace.HBM),
            scratch_shapes=[
                pltpu.VMEM((B, N, C), jnp.float32),
                pltpu.VMEM((B, N, C), jnp.float32),
                pltpu.VMEM((B * K, C, H), jnp.float32),
                pltpu.VMEM((B * K, H, C), jnp.float32),
                pltpu.SemaphoreType.DMA((B * _NCH,)),
                pltpu.SemaphoreType.DMA((2 * B * K,)),
                pltpu.SemaphoreType.DMA((B * _NCH,)),
            ],
        ),
        out_shape=jax.ShapeDtypeStruct((B, N, C), jnp.float32),
        compiler_params=pltpu.CompilerParams(
            dimension_semantics=("arbitrary",),
        ),
    )(task_ids, x, gate_w, epsT, w1, b1, w2, b2)
    return out


# speedup vs baseline: 1.0295x; 1.0295x over previous
import jax
import jax.numpy as jnp
from jax.experimental import pallas as pl
from jax.experimental.pallas import tpu as pltpu

B, N, C = 2, 2048, 768
E, H, D, K = 8, 384, 4, 2

_NEG_INF = float("-inf")
_CH = 2048
_NCH = N // _CH


def _gelu(v):
    return v * 0.5 * (1.0 + jax.lax.erf(v * 0.7071067811865476))


def _moe_kernel(tids_ref, x_hbm, gw_ref, epsT_ref, w1_hbm, b1_ref, w2_hbm,
                b2_ref, out_hbm, xv, outv, w1s, w2s, sx, sw, so):
    cps_x = []
    for b in range(B):
        for i in range(_NCH):
            cp = pltpu.make_async_copy(
                x_hbm.at[b, pl.ds(i * _CH, _CH), :],
                xv.at[b, pl.ds(i * _CH, _CH), :],
                sx.at[b * _NCH + i])
            cp.start()
            cps_x.append(cp)

    gate_info = []
    for b in range(B):
        tid = tids_ref[b]
        gwb = gw_ref[pl.ds(tid, 1), :, :][0]
        s = jnp.zeros((1, E), jnp.float32)
        for i in range(_NCH):
            cps_x[b * _NCH + i].wait()
            xc = xv[b, pl.ds(i * _CH, _CH), :]
            twc = jnp.dot(xc, gwb, preferred_element_type=jnp.float32)
            clean = twc[:, :E]
            stdc = jax.nn.softplus(twc[:, E:]) + 0.01
            epc = epsT_ref[b, :, pl.ds(i * _CH, _CH)]
            m = jnp.dot(epc, stdc, preferred_element_type=jnp.float32)
            ir = jax.lax.broadcasted_iota(jnp.int32, (E, E), 0)
            ic = jax.lax.broadcasted_iota(jnp.int32, (E, E), 1)
            diag = jnp.sum(jnp.where(ir == ic, m, 0.0), axis=0, keepdims=True)
            s = s + jnp.sum(clean, axis=0, keepdims=True) + diag
        iota = jax.lax.broadcasted_iota(jnp.int32, (1, E), 1)
        m2 = jnp.max(s)
        e0 = jnp.min(jnp.where(s == m2, iota, E))
        masked = jnp.where(iota == e0, _NEG_INF, s)
        m1 = jnp.max(masked)
        e1 = jnp.min(jnp.where(masked == m1, iota, E))
        d = m2 - m1
        a = d / (d + 1e-6)
        ena = jnp.exp(-a)
        g0 = 1.0 / (1.0 + ena)
        g1 = ena / (1.0 + ena)
        cws = []
        for j, e in enumerate((e0, e1)):
            cp1 = pltpu.make_async_copy(w1_hbm.at[e], w1s.at[b * K + j],
                                        sw.at[b * K + j])
            cp2 = pltpu.make_async_copy(w2_hbm.at[e], w2s.at[b * K + j],
                                        sw.at[B * K + b * K + j])
            cp1.start()
            cp2.start()
            cws.append((cp1, cp2))
        gate_info.append((e0, e1, g0, g1, cws))

    cps_o = []
    for b in range(B):
        e0, e1, g0, g1, cws = gate_info[b]
        cws[0][0].wait()
        cws[0][1].wait()
        cws[1][0].wait()
        cws[1][1].wait()
        w1cat = jnp.concatenate(
            [w1s[b * K + 0], w1s[b * K + 1]], axis=1).astype(jnp.bfloat16)
        w2cat = jnp.concatenate(
            [w2s[b * K + 0], w2s[b * K + 1]], axis=0).astype(jnp.bfloat16)
        b1cat = jnp.concatenate(
            [b1_ref[pl.ds(e0, 1), :], b1_ref[pl.ds(e1, 1), :]], axis=1)
        gsc = jnp.concatenate(
            [jnp.full((1, H), 1.0, jnp.float32) * g0,
             jnp.full((1, H), 1.0, jnp.float32) * g1], axis=1)
        b2mix = g0 * b2_ref[pl.ds(e0, 1), :] + g1 * b2_ref[pl.ds(e1, 1), :]
        for i in range(_NCH):
            xc = xv[b, pl.ds(i * _CH, _CH), :]
            xbf = xc.astype(jnp.bfloat16)
            h = jnp.dot(xbf, w1cat, preferred_element_type=jnp.float32) + b1cat
            hg = (_gelu(h) * gsc).astype(jnp.bfloat16)
            y = jnp.dot(hg, w2cat, preferred_element_type=jnp.float32) + b2mix
            outv[b, pl.ds(i * _CH, _CH), :] = xc + y
            cp = pltpu.make_async_copy(
                outv.at[b, pl.ds(i * _CH, _CH), :],
                out_hbm.at[b, pl.ds(i * _CH, _CH), :],
                so.at[b * _NCH + i])
            cp.start()
            cps_o.append(cp)

    for cp in cps_o:
        cp.wait()


@jax.jit
def kernel(x, gate_w, w1, b1, w2, b2, eps, task_ids):
    task_ids = task_ids.astype(jnp.int32)
    epsT = jnp.transpose(eps, (0, 2, 1))

    out = pl.pallas_call(
        _moe_kernel,
        grid_spec=pltpu.PrefetchScalarGridSpec(
            num_scalar_prefetch=1,
            grid=(1,),
            in_specs=[
                pl.BlockSpec(memory_space=pltpu.MemorySpace.HBM),
                pl.BlockSpec((D, C, 2 * E), lambda i, tids: (0, 0, 0)),
                pl.BlockSpec((B, E, N), lambda i, tids: (0, 0, 0)),
                pl.BlockSpec(memory_space=pltpu.MemorySpace.HBM),
                pl.BlockSpec((E, H), lambda i, tids: (0, 0)),
                pl.BlockSpec(memory_space=pltpu.MemorySpace.HBM),
                pl.BlockSpec((E, C), lambda i, tids: (0, 0)),
            ],
            out_specs=pl.BlockSpec(memory_space=pltpu.MemorySpace.HBM),
            scratch_shapes=[
                pltpu.VMEM((B, N, C), jnp.float32),
                pltpu.VMEM((B, N, C), jnp.float32),
                pltpu.VMEM((B * K, C, H), jnp.float32),
                pltpu.VMEM((B * K, H, C), jnp.float32),
                pltpu.SemaphoreType.DMA((B * _NCH,)),
                pltpu.SemaphoreType.DMA((2 * B * K,)),
                pltpu.SemaphoreType.DMA((B * _NCH,)),
            ],
        ),
        out_shape=jax.ShapeDtypeStruct((B, N, C), jnp.float32),
        compiler_params=pltpu.CompilerParams(
            dimension_semantics=("arbitrary",),
        ),
    )(task_ids, x, gate_w, epsT, w1, b1, w2, b2)
    return out


# speedup vs baseline: 1.1308x; 1.0984x over previous
import jax
import jax.numpy as jnp
from jax.experimental import pallas as pl
from jax.experimental.pallas import tpu as pltpu

B, N, C = 2, 2048, 768
E, H, D, K = 8, 384, 4, 2

_NEG_INF = float("-inf")
_CH = 1024
_NCH = N // _CH


def _gelu(v):
    return v * 0.5 * (1.0 + jax.lax.erf(v * 0.7071067811865476))


def _moe_kernel(tids_ref, x_hbm, gw_ref, epsT_ref, w1_hbm, b1_ref, w2_hbm,
                b2_ref, out_hbm, xv, outv, w1s, w2s, sx, sw, so):
    cps_x = []
    for b in range(B):
        for i in range(_NCH):
            cp = pltpu.make_async_copy(
                x_hbm.at[b, pl.ds(i * _CH, _CH), :],
                xv.at[b, pl.ds(i * _CH, _CH), :],
                sx.at[b * _NCH + i])
            cp.start()
            cps_x.append(cp)

    gate_info = []
    for b in range(B):
        tid = tids_ref[b]
        gwb = gw_ref[pl.ds(tid, 1), :, :][0]
        s = jnp.zeros((1, E), jnp.float32)
        for i in range(_NCH):
            cps_x[b * _NCH + i].wait()
            xc = xv[b, pl.ds(i * _CH, _CH), :]
            twc = jnp.dot(xc, gwb, preferred_element_type=jnp.float32)
            clean = twc[:, :E]
            stdc = jax.nn.softplus(twc[:, E:]) + 0.01
            epc = epsT_ref[b, :, pl.ds(i * _CH, _CH)]
            m = jnp.dot(epc, stdc, preferred_element_type=jnp.float32)
            ir = jax.lax.broadcasted_iota(jnp.int32, (E, E), 0)
            ic = jax.lax.broadcasted_iota(jnp.int32, (E, E), 1)
            diag = jnp.sum(jnp.where(ir == ic, m, 0.0), axis=0, keepdims=True)
            s = s + jnp.sum(clean, axis=0, keepdims=True) + diag
        iota = jax.lax.broadcasted_iota(jnp.int32, (1, E), 1)
        m2 = jnp.max(s)
        e0 = jnp.min(jnp.where(s == m2, iota, E))
        masked = jnp.where(iota == e0, _NEG_INF, s)
        m1 = jnp.max(masked)
        e1 = jnp.min(jnp.where(masked == m1, iota, E))
        d = m2 - m1
        a = d / (d + 1e-6)
        ena = jnp.exp(-a)
        g0 = 1.0 / (1.0 + ena)
        g1 = ena / (1.0 + ena)
        cws = []
        for j, e in enumerate((e0, e1)):
            cp1 = pltpu.make_async_copy(w1_hbm.at[e], w1s.at[b * K + j],
                                        sw.at[b * K + j])
            cp2 = pltpu.make_async_copy(w2_hbm.at[e], w2s.at[b * K + j],
                                        sw.at[B * K + b * K + j])
            cp1.start()
            cp2.start()
            cws.append((cp1, cp2))
        gate_info.append((e0, e1, g0, g1, cws))

    cps_o = []
    for b in range(B):
        e0, e1, g0, g1, cws = gate_info[b]
        cws[0][0].wait()
        cws[0][1].wait()
        cws[1][0].wait()
        cws[1][1].wait()
        w1cat = jnp.concatenate(
            [w1s[b * K + 0], w1s[b * K + 1]], axis=1).astype(jnp.bfloat16)
        w2cat = jnp.concatenate(
            [w2s[b * K + 0], w2s[b * K + 1]], axis=0).astype(jnp.bfloat16)
        b1cat = jnp.concatenate(
            [b1_ref[pl.ds(e0, 1), :], b1_ref[pl.ds(e1, 1), :]], axis=1)
        gsc = jnp.concatenate(
            [jnp.full((1, H), 1.0, jnp.float32) * g0,
             jnp.full((1, H), 1.0, jnp.float32) * g1], axis=1)
        b2mix = g0 * b2_ref[pl.ds(e0, 1), :] + g1 * b2_ref[pl.ds(e1, 1), :]
        for i in range(_NCH):
            xc = xv[b, pl.ds(i * _CH, _CH), :]
            xbf = xc.astype(jnp.bfloat16)
            h = jnp.dot(xbf, w1cat, preferred_element_type=jnp.float32) + b1cat
            hg = (_gelu(h) * gsc).astype(jnp.bfloat16)
            y = jnp.dot(hg, w2cat, preferred_element_type=jnp.float32) + b2mix
            outv[b, pl.ds(i * _CH, _CH), :] = xc + y
            cp = pltpu.make_async_copy(
                outv.at[b, pl.ds(i * _CH, _CH), :],
                out_hbm.at[b, pl.ds(i * _CH, _CH), :],
                so.at[b * _NCH + i])
            cp.start()
            cps_o.append(cp)

    for cp in cps_o:
        cp.wait()


@jax.jit
def kernel(x, gate_w, w1, b1, w2, b2, eps, task_ids):
    task_ids = task_ids.astype(jnp.int32)
    epsT = jnp.transpose(eps, (0, 2, 1))

    out = pl.pallas_call(
        _moe_kernel,
        grid_spec=pltpu.PrefetchScalarGridSpec(
            num_scalar_prefetch=1,
            grid=(1,),
            in_specs=[
                pl.BlockSpec(memory_space=pltpu.MemorySpace.HBM),
                pl.BlockSpec((D, C, 2 * E), lambda i, tids: (0, 0, 0)),
                pl.BlockSpec((B, E, N), lambda i, tids: (0, 0, 0)),
                pl.BlockSpec(memory_space=pltpu.MemorySpace.HBM),
                pl.BlockSpec((E, H), lambda i, tids: (0, 0)),
                pl.BlockSpec(memory_space=pltpu.MemorySpace.HBM),
                pl.BlockSpec((E, C), lambda i, tids: (0, 0)),
            ],
            out_specs=pl.BlockSpec(memory_space=pltpu.MemorySpace.HBM),
            scratch_shapes=[
                pltpu.VMEM((B, N, C), jnp.float32),
                pltpu.VMEM((B, N, C), jnp.float32),
                pltpu.VMEM((B * K, C, H), jnp.float32),
                pltpu.VMEM((B * K, H, C), jnp.float32),
                pltpu.SemaphoreType.DMA((B * _NCH,)),
                pltpu.SemaphoreType.DMA((2 * B * K,)),
                pltpu.SemaphoreType.DMA((B * _NCH,)),
            ],
        ),
        out_shape=jax.ShapeDtypeStruct((B, N, C), jnp.float32),
        compiler_params=pltpu.CompilerParams(
            dimension_semantics=("arbitrary",),
        ),
    )(task_ids, x, gate_w, epsT, w1, b1, w2, b2)
    return out
